# Initial kernel scaffold; baseline (speedup 1.0000x reference)
import jax
import jax.numpy as jnp
from jax.experimental import pallas as pl


def kernel(cosine, label):
    raise NotImplementedError("write your pallas kernel here")


# speedup vs baseline: 3.0549x; 3.0549x over previous
import math

import jax
import jax.numpy as jnp
from jax.experimental import pallas as pl
from jax.experimental.pallas import tpu as pltpu

_S = 64.0
_M2 = 0.3
_M3 = 0.2
_COS_M2 = math.cos(_M2)
_SIN_M2 = math.sin(_M2)


def _body(bn, lab_ref, x_ref, o_ref):
    j = pl.program_id(1)
    x = x_ref[...]
    lab = lab_ref[...]
    col = jax.lax.broadcasted_iota(jnp.int32, x.shape, 1) + j * bn
    t = jnp.maximum(1.0 - x * x, 0.0)
    fix = (x * _COS_M2 - jnp.sqrt(t) * _SIN_M2 - _M3) * _S
    o_ref[...] = jnp.where(col == lab, fix, x * _S)


def kernel(cosine, label):
    B, C = cosine.shape
    bm, bn = 256, 4096
    grid = (pl.cdiv(B, bm), pl.cdiv(C, bn))
    lab2 = label.reshape(B, 1)
    import functools
    return pl.pallas_call(
        functools.partial(_body, bn),
        grid=grid,
        in_specs=[
            pl.BlockSpec((bm, 1), lambda i, j: (i, 0)),
            pl.BlockSpec((bm, bn), lambda i, j: (i, j)),
        ],
        out_specs=pl.BlockSpec((bm, bn), lambda i, j: (i, j)),
        out_shape=jax.ShapeDtypeStruct((B, C), cosine.dtype),
        compiler_params=pltpu.CompilerParams(
            dimension_semantics=("parallel", "parallel"),
        ),
    )(lab2, cosine)


# speedup vs baseline: 3.3950x; 1.1113x over previous
import functools
import math

import jax
import jax.numpy as jnp
from jax import lax
from jax.experimental import pallas as pl
from jax.experimental.pallas import tpu as pltpu
from jax.experimental.pallas import tpu_sc as plsc

_S = 64.0
_M2 = 0.3
_M3 = 0.2
_COS_M2 = math.cos(_M2)
_SIN_M2 = math.sin(_M2)

_NC, _NS, _L = 2, 16, 16
_NW = _NC * _NS


def _sc_gather_body(bpw, cos_hbm, lab_hbm, xg_hbm, labv, xall, xstage, sem):
    wid = lax.axis_index("s") * _NC + lax.axis_index("c")
    base = wid * bpw
    pltpu.sync_copy(lab_hbm.at[pl.ds(base, bpw)], labv)
    lanes = lax.iota(jnp.int32, _L)
    copies = []
    for k in range(bpw // _L):
        lab16 = labv[pl.ds(k * _L, _L)]
        for j in range(_L):
            r = k * _L + j
            lab_r = lab16[j]
            col0 = pl.multiple_of(lab_r & jnp.int32(-128), 128)
            row0 = pl.multiple_of(base + (r & ~7), 8)
            copies.append(
                pltpu.async_copy(
                    cos_hbm.at[pl.ds(row0, 8), pl.ds(col0, 128)],
                    xall.at[r],
                    sem,
                )
            )
    for cp in copies:
        cp.wait()
    for k in range(bpw // _L):
        lab16 = labv[pl.ds(k * _L, _L)]
        for j in range(_L):
            r = k * _L + j
            lab_r = lab16[j]
            start = pl.multiple_of((lab_r & 127) & jnp.int32(-_L), _L)
            wv = xall[r, r & 7, pl.ds(start, _L)]
            mask = lanes == (lab_r & (_L - 1))
            plsc.store_scatter(
                xstage, [jnp.full((_L,), r, jnp.int32)], wv, mask=mask
            )
    pltpu.sync_copy(xstage, xg_hbm.at[pl.ds(base, bpw)])


def _sc_gather(cosine, label):
    B, C = cosine.shape
    bpw = B // _NW
    mesh = plsc.VectorSubcoreMesh(core_axis_name="c", subcore_axis_name="s")
    return pl.kernel(
        functools.partial(_sc_gather_body, bpw),
        out_type=jax.ShapeDtypeStruct((B,), jnp.float32),
        mesh=mesh,
        scratch_types=[
            pltpu.VMEM((bpw,), jnp.int32),
            pltpu.VMEM((bpw, 8, 128), jnp.float32),
            pltpu.VMEM((bpw,), jnp.float32),
            pltpu.SemaphoreType.DMA,
        ],
        compiler_params=pltpu.CompilerParams(needs_layout_passes=False),
    )(cosine, label)


def _tc_body(bn, lab_ref, xg_ref, x_ref, o_ref):
    j = pl.program_id(1)
    x = x_ref[...]
    lab = lab_ref[...]
    xg = xg_ref[...]
    t = jnp.maximum(1.0 - xg * xg, 0.0)
    fix = (xg * _COS_M2 - jnp.sqrt(t) * _SIN_M2 - _M3) * _S
    col = jax.lax.broadcasted_iota(jnp.int32, x.shape, 1) + j * bn
    o_ref[...] = jnp.where(col == lab, fix, x * _S)


def kernel(cosine, label):
    B, C = cosine.shape
    xg = _sc_gather(cosine, label)
    bm, bn = 256, 4096
    grid = (pl.cdiv(B, bm), pl.cdiv(C, bn))
    lab2 = label.reshape(B, 1)
    xg2 = xg.reshape(B, 1)
    return pl.pallas_call(
        functools.partial(_tc_body, bn),
        grid=grid,
        in_specs=[
            pl.BlockSpec((bm, 1), lambda i, j: (i, 0)),
            pl.BlockSpec((bm, 1), lambda i, j: (i, 0)),
            pl.BlockSpec((bm, bn), lambda i, j: (i, j)),
        ],
        out_specs=pl.BlockSpec((bm, bn), lambda i, j: (i, j)),
        out_shape=jax.ShapeDtypeStruct((B, C), cosine.dtype),
        compiler_params=pltpu.CompilerParams(
            dimension_semantics=("parallel", "parallel"),
        ),
    )(lab2, xg2, cosine)


# speedup vs baseline: 3.4079x; 1.0038x over previous
import functools
import math

import jax
import jax.numpy as jnp
from jax import lax
from jax.experimental import pallas as pl
from jax.experimental.pallas import tpu as pltpu
from jax.experimental.pallas import tpu_sc as plsc

_S = 64.0
_M2 = 0.3
_M3 = 0.2
_COS_M2 = math.cos(_M2)
_SIN_M2 = math.sin(_M2)

_NC, _NS, _L = 2, 16, 16
_NW = _NC * _NS


def _sc_gather_body(bpw, cos_hbm, lab_hbm, xg_hbm, labv, xall, xstage, sem):
    wid = lax.axis_index("s") * _NC + lax.axis_index("c")
    base = wid * bpw
    pltpu.sync_copy(lab_hbm.at[pl.ds(base, bpw)], labv)
    lanes = lax.iota(jnp.int32, _L)
    copies = []
    for k in range(bpw // _L):
        lab16 = labv[pl.ds(k * _L, _L)]
        for j in range(_L):
            r = k * _L + j
            lab_r = lab16[j]
            col0 = pl.multiple_of(lab_r & jnp.int32(-128), 128)
            row0 = pl.multiple_of(base + (r & ~7), 8)
            copies.append(
                pltpu.async_copy(
                    cos_hbm.at[pl.ds(row0, 8), pl.ds(col0, 128)],
                    xall.at[r],
                    sem,
                )
            )
    for cp in copies:
        cp.wait()
    for k in range(bpw // _L):
        lab16 = labv[pl.ds(k * _L, _L)]
        for j in range(_L):
            r = k * _L + j
            lab_r = lab16[j]
            start = pl.multiple_of((lab_r & 127) & jnp.int32(-_L), _L)
            wv = xall[r, r & 7, pl.ds(start, _L)]
            mask = lanes == (lab_r & (_L - 1))
            plsc.store_scatter(
                xstage, [jnp.full((_L,), r, jnp.int32)], wv, mask=mask
            )
    pltpu.sync_copy(xstage, xg_hbm.at[pl.ds(base, bpw)])


def _sc_gather(cosine, label):
    B, C = cosine.shape
    bpw = B // _NW
    mesh = plsc.VectorSubcoreMesh(core_axis_name="c", subcore_axis_name="s")
    return pl.kernel(
        functools.partial(_sc_gather_body, bpw),
        out_type=jax.ShapeDtypeStruct((B,), jnp.float32),
        mesh=mesh,
        scratch_types=[
            pltpu.VMEM((bpw,), jnp.int32),
            pltpu.VMEM((bpw, 8, 128), jnp.float32),
            pltpu.VMEM((bpw,), jnp.float32),
            pltpu.SemaphoreType.DMA,
        ],
        compiler_params=pltpu.CompilerParams(needs_layout_passes=False),
    )(cosine, label)


def _tc_body(bn, lab_ref, xg_ref, x_ref, o_ref):
    j = pl.program_id(1)
    x = x_ref[...]
    lab = lab_ref[...]
    xg = xg_ref[...]
    t = jnp.maximum(1.0 - xg * xg, 0.0)
    fix = (xg * _COS_M2 - jnp.sqrt(t) * _SIN_M2 - _M3) * _S
    col = jax.lax.broadcasted_iota(jnp.int32, x.shape, 1) + j * bn
    o_ref[...] = jnp.where(col == lab, fix, x * _S)


def kernel(cosine, label):
    B, C = cosine.shape
    xg = _sc_gather(cosine, label)
    bm, bn = 256, 8192
    grid = (pl.cdiv(B, bm), pl.cdiv(C, bn))
    lab2 = label.reshape(B, 1)
    xg2 = xg.reshape(B, 1)
    return pl.pallas_call(
        functools.partial(_tc_body, bn),
        grid=grid,
        in_specs=[
            pl.BlockSpec((bm, 1), lambda i, j: (i, 0)),
            pl.BlockSpec((bm, 1), lambda i, j: (i, 0)),
            pl.BlockSpec((bm, bn), lambda i, j: (i, j)),
        ],
        out_specs=pl.BlockSpec((bm, bn), lambda i, j: (i, j)),
        out_shape=jax.ShapeDtypeStruct((B, C), cosine.dtype),
        compiler_params=pltpu.CompilerParams(
            dimension_semantics=("parallel", "parallel"),
        ),
    )(lab2, xg2, cosine)
